# baseline (speedup 1.0000x reference)
import jax
import jax.numpy as jnp
from jax import lax
from jax.experimental import pallas as pl
from jax.experimental.pallas import tpu as pltpu
from jax.experimental.pallas import tpu_sc as plsc

N_NODES = 10000
N_EDGES = 320000
D_FEAT = 128
EMBED_DIM = 128
NUM_CLASSES = 40
BATCH = 1024

NC = 2
NS = 16
NW = NC * NS
EPW = N_EDGES // NW
EVECS = EPW // 16
MAP_PAD = 10016
CHUNK = 128
LIST_ROWS = (EPW + 127) // CHUNK + 1
ACC_ROWS = BATCH + CHUNK
ZROWS = ACC_ROWS // NS
DROWS = 8
DCOLS = 128
BPS = BATCH // NS
NBUF = 3


def _sc_body(x_hbm, src_hbm, dst_hbm, nodes_hbm, mneg_hbm, zacc_hbm,
             zdeg_hbm, accpos_hbm, degpos_hbm, xg_hbm,
             map_v, nodes_v, dst_v, src_v, srclist_v, slotlist_v,
             rows_v, deg_v, degtmp_v, degfin_v, degall_v, accsel_v,
             degsel_v, slotsel_v, selfrows_v, acc_sh, degstage_sh,
             degfinal_sh, dsem, esem, xsem):
    c = lax.axis_index("c")
    s = lax.axis_index("s")
    w = s * NC + c
    iota16 = lax.iota(jnp.int32, 16)

    pltpu.sync_copy(nodes_hbm, nodes_v)
    pltpu.sync_copy(mneg_hbm, map_v)
    pltpu.sync_copy(zdeg_hbm, deg_v)

    cp_dst = pltpu.async_copy(dst_hbm.at[pl.ds(w * EPW, EPW)], dst_v, esem)
    cp_src = pltpu.async_copy(src_hbm.at[pl.ds(w * EPW, EPW)], src_v, esem)

    pltpu.sync_copy(zacc_hbm.at[pl.ds(s * ZROWS, ZROWS)],
                    acc_sh.at[pl.ds(s * ZROWS, ZROWS)])

    def _mapbuild(i, carry):
        nd = nodes_v[pl.ds(i * 16, 16)]
        plsc.store_scatter(map_v, [nd], i * 16 + iota16)
        return carry

    lax.fori_loop(0, BATCH // 16, _mapbuild, 0)

    @pl.when(c == 0)
    def _selfgather():
        pltpu.async_copy(
            x_hbm.at[nodes_v.at[pl.ds(s * BPS, BPS)]], selfrows_v, xsem)

    cp_dst.wait()
    cp_src.wait()

    ones16 = jnp.ones((16,), jnp.float32)

    def _scan(i, nvec):
        d = dst_v[pl.ds(i * 16, 16)]
        sv = src_v[pl.ds(i * 16, 16)]
        slot = plsc.load_gather(map_v, [d])
        m = slot >= 0
        plsc.addupdate_scatter(
            deg_v,
            [lax.shift_right_logical(slot, 7), lax.bitwise_and(slot, 127)],
            ones16, mask=m)
        inc = jnp.where(m, 1, 0).astype(jnp.int32)
        pos = nvec + plsc.cumsum(inc) - 1
        r = lax.shift_right_logical(pos, 7)
        cc = lax.bitwise_and(pos, 127)
        plsc.store_scatter(srclist_v, [r, cc], sv, mask=m)
        plsc.store_scatter(slotlist_v, [r, cc], slot, mask=m)
        return nvec + plsc.all_reduce_population_count(m)

    with jax.named_scope("edge_scan"):
        nvec = lax.fori_loop(0, EVECS, _scan, jnp.zeros((16,), jnp.int32))
    n_valid = jnp.max(nvec)

    dummy_slot = jnp.full((16,), BATCH, jnp.int32)
    zero16 = jnp.zeros((16,), jnp.int32)
    for k in range(CHUNK // 16):
        p = n_valid + k * 16 + iota16
        r = lax.shift_right_logical(p, 7)
        cc = lax.bitwise_and(p, 127)
        plsc.store_scatter(srclist_v, [r, cc], zero16)
        plsc.store_scatter(slotlist_v, [r, cc], dummy_slot)

    pltpu.sync_copy(deg_v, degstage_sh.at[s])

    with jax.named_scope("barrier1"):
        plsc.subcore_barrier()

    nch = lax.div(n_valid + (CHUNK - 1), jnp.int32(CHUNK))
    total = lax.max(nch, jnp.int32(1))

    def _grp(g, carry):
        j0 = g * NBUF
        for b in range(NBUF):
            @pl.when(j0 + b < total)
            def _fire():
                pltpu.async_copy(
                    x_hbm.at[srclist_v.at[j0 + b]], rows_v.at[b], dsem)
        for b in range(NBUF):
            @pl.when(j0 + b < total)
            def _drain():
                pltpu.make_async_copy(
                    x_hbm.at[srclist_v.at[j0 + b]], rows_v.at[b],
                    dsem).wait()
        for b in range(NBUF):
            @pl.when(j0 + b < total)
            def _scat():
                pltpu.sync_copy(
                    rows_v.at[b], acc_sh.at[slotlist_v.at[j0 + b]],
                    add=True)
        return carry

    with jax.named_scope("chunk_loop"):
        lax.fori_loop(
            0, lax.div(total + (NBUF - 1), jnp.int32(NBUF)), _grp, 0)

    r0 = lax.shift_right_logical(s, 1)
    cb = lax.bitwise_and(s, 1) * 64
    pltpu.sync_copy(degstage_sh.at[:, r0, :], degtmp_v)

    def _red(t, a):
        return (a[0] + degtmp_v[t, pl.ds(cb, 16)],
                a[1] + degtmp_v[t, pl.ds(cb + 16, 16)],
                a[2] + degtmp_v[t, pl.ds(cb + 32, 16)],
                a[3] + degtmp_v[t, pl.ds(cb + 48, 16)])

    z16 = jnp.zeros((16,), jnp.float32)
    a = lax.fori_loop(0, NS, _red, (z16, z16, z16, z16))
    for k in range(4):
        degfin_v[pl.ds(k * 16, 16)] = a[k]
    pltpu.sync_copy(degfin_v, degfinal_sh.at[r0, pl.ds(cb, 64)])

    with jax.named_scope("barrier2"):
        plsc.subcore_barrier()

    pltpu.sync_copy(degfinal_sh, degall_v)

    def _slots(k, carry):
        nd = nodes_v[pl.ds(s * BPS + k * 16, 16)]
        sl = plsc.load_gather(map_v, [nd])
        slotsel_v[pl.ds(k * 16, 16)] = sl
        degsel_v[pl.ds(k * 16, 16)] = plsc.load_gather(
            degall_v,
            [lax.shift_right_logical(sl, 7), lax.bitwise_and(sl, 127)])
        return carry

    lax.fori_loop(0, BPS // 16, _slots, 0)

    for half in range(2):
        pltpu.async_copy(
            acc_sh.at[slotsel_v.at[pl.ds(half * 32, 32)]], accsel_v,
            dsem).wait()
        pltpu.sync_copy(
            accsel_v, accpos_hbm.at[c, pl.ds(s * BPS + half * 32, 32)])
    pltpu.sync_copy(degsel_v, degpos_hbm.at[c, pl.ds(s * BPS, BPS)])

    @pl.when(c == 0)
    def _selfwrite():
        pltpu.make_async_copy(
            x_hbm.at[nodes_v.at[pl.ds(s * BPS, BPS)]], selfrows_v,
            xsem).wait()
        pltpu.sync_copy(selfrows_v, xg_hbm.at[pl.ds(s * BPS, BPS)])


def _sc_stage(x, src, dst, nodes32):
    mneg = jnp.full((MAP_PAD,), -1, jnp.int32)
    zacc = jnp.zeros((ACC_ROWS, D_FEAT), jnp.float32)
    zdeg = jnp.zeros((DROWS, DCOLS), jnp.float32)
    mesh = plsc.VectorSubcoreMesh(
        core_axis_name="c", subcore_axis_name="s",
        num_cores=NC, num_subcores=NS)
    return pl.kernel(
        _sc_body,
        out_type=[
            jax.ShapeDtypeStruct((NC, BATCH, D_FEAT), jnp.float32),
            jax.ShapeDtypeStruct((NC, BATCH), jnp.float32),
            jax.ShapeDtypeStruct((BATCH, D_FEAT), jnp.float32),
        ],
        mesh=mesh,
        compiler_params=pltpu.CompilerParams(needs_layout_passes=False),
        scratch_types=[
            pltpu.VMEM((MAP_PAD,), jnp.int32),
            pltpu.VMEM((BATCH,), jnp.int32),
            pltpu.VMEM((EPW,), jnp.int32),
            pltpu.VMEM((EPW,), jnp.int32),
            pltpu.VMEM((LIST_ROWS, CHUNK), jnp.int32),
            pltpu.VMEM((LIST_ROWS, CHUNK), jnp.int32),
            pltpu.VMEM((NBUF, CHUNK, D_FEAT), jnp.float32),
            pltpu.VMEM((DROWS, DCOLS), jnp.float32),
            pltpu.VMEM((NS, DCOLS), jnp.float32),
            pltpu.VMEM((64,), jnp.float32),
            pltpu.VMEM((DROWS, DCOLS), jnp.float32),
            pltpu.VMEM((32, D_FEAT), jnp.float32),
            pltpu.VMEM((BPS,), jnp.float32),
            pltpu.VMEM((BPS,), jnp.int32),
            pltpu.VMEM((BPS, D_FEAT), jnp.float32),
            pltpu.VMEM_SHARED((ACC_ROWS, D_FEAT), jnp.float32),
            pltpu.VMEM_SHARED((NS, DROWS, DCOLS), jnp.float32),
            pltpu.VMEM_SHARED((DROWS, DCOLS), jnp.float32),
            pltpu.SemaphoreType.DMA,
            pltpu.SemaphoreType.DMA,
            pltpu.SemaphoreType.DMA,
        ],
    )(x, src, dst, nodes32, mneg, zacc, zdeg)


def _tc_body(accpos_ref, degpos_ref, xg_ref, wenc_ref, wcls_ref, b_ref,
             out_ref):
    acc = accpos_ref[0] + accpos_ref[1]
    deg = degpos_ref[0] + degpos_ref[1]
    neigh = acc / jnp.maximum(deg, 1.0)[:, None]
    w_self = wenc_ref[:, :D_FEAT]
    w_neigh = wenc_ref[:, D_FEAT:]
    dn = (((1,), (1,)), ((), ()))
    h = lax.dot_general(xg_ref[...], w_self, dn,
                        preferred_element_type=jnp.float32)
    h += lax.dot_general(neigh, w_neigh, dn,
                         preferred_element_type=jnp.float32)
    h = jnp.maximum(h, 0.0)
    out_ref[...] = lax.dot_general(h, wcls_ref[...], dn,
                                   preferred_element_type=jnp.float32) + b_ref[...]


def _tc_stage(accpos, degpos, xg, W_enc, W_cls, b2):
    return pl.pallas_call(
        _tc_body,
        out_shape=jax.ShapeDtypeStruct((BATCH, NUM_CLASSES), jnp.float32),
    )(accpos, degpos, xg, W_enc, W_cls, b2)


def kernel(x, edge_index, nodes, W_enc, W_cls, b_cls):
    src = edge_index[0].astype(jnp.int32)
    dst = edge_index[1].astype(jnp.int32)
    nodes32 = nodes.astype(jnp.int32)
    accpos, degpos, xg = _sc_stage(x, src, dst, nodes32)
    b2 = b_cls.reshape(1, NUM_CLASSES)
    return _tc_stage(accpos, degpos, xg, W_enc, W_cls, b2)


# speedup vs baseline: 1.4551x; 1.4551x over previous
import jax
import jax.numpy as jnp
from jax import lax
from jax.experimental import pallas as pl
from jax.experimental.pallas import tpu as pltpu
from jax.experimental.pallas import tpu_sc as plsc

N_NODES = 10000
N_EDGES = 320000
D_FEAT = 128
EMBED_DIM = 128
NUM_CLASSES = 40
BATCH = 1024

NC = 2
NS = 16
NW = NC * NS
EPW = N_EDGES // NW
NSEG = 5
SEGE = EPW // NSEG
SVECS = SEGE // 16
CHUNK = 128
PROWS = SEGE // CHUNK + 2
ACC_ROWS = BATCH + CHUNK
ZROWS = ACC_ROWS // NS
XROWS = 632
XTAIL = N_NODES - (NS - 1) * XROWS
DROWS = 8
DCOLS = 128
BPS = BATCH // NS
PACK = 2048


def _sc_body(x_hbm, src_hbm, dst_hbm, nodes_hbm, mneg_hbm, zacc_hbm,
             zdeg_hbm, accpos_hbm, degpos_hbm, xg_hbm,
             map_v, nodes_v, dst_v, src_v, pair_v, rows_v, srcidx_v,
             slotidx_v, deg_v, degtmp_v, degfin_v, degall_v, slotsel_v,
             degsel_v, x_sh, acc_sh, degstage_sh, degfinal_sh, dsem):
    c = lax.axis_index("c")
    s = lax.axis_index("s")
    w = s * NC + c
    iota16 = lax.iota(jnp.int32, 16)

    pltpu.sync_copy(nodes_hbm, nodes_v)
    pltpu.sync_copy(mneg_hbm, map_v)
    pltpu.sync_copy(zdeg_hbm, deg_v)

    pltpu.sync_copy(zacc_hbm.at[pl.ds(s * ZROWS, ZROWS)],
                    acc_sh.at[pl.ds(s * ZROWS, ZROWS)])
    @pl.when(s < NS - 1)
    def _xstage():
        pltpu.sync_copy(x_hbm.at[pl.ds(s * XROWS, XROWS)],
                        x_sh.at[pl.ds(s * XROWS, XROWS)])

    @pl.when(s == NS - 1)
    def _xstage_tail():
        pltpu.sync_copy(x_hbm.at[pl.ds((NS - 1) * XROWS, XTAIL)],
                        x_sh.at[pl.ds((NS - 1) * XROWS, XTAIL)])

    def _mapbuild(i, carry):
        nd = nodes_v[pl.ds(i * 16, 16)]
        plsc.store_scatter(map_v, [nd], i * 16 + iota16)
        return carry

    lax.fori_loop(0, BATCH // 16, _mapbuild, 0)

    plsc.subcore_barrier()

    ones16 = jnp.ones((16,), jnp.float32)
    dummy_pair = jnp.full((16,), BATCH, jnp.int32)

    def _chunk(j, carry):
        for k in range(CHUNK // 16):
            pv = pair_v[j, pl.ds(k * 16, 16)]
            srcidx_v[0, pl.ds(k * 16, 16)] = lax.shift_right_logical(pv, 11)
            slotidx_v[0, pl.ds(k * 16, 16)] = lax.bitwise_and(
                pv, PACK - 1)
        pltpu.async_copy(x_sh.at[srcidx_v.at[0]], rows_v, dsem).wait()
        pltpu.sync_copy(rows_v, acc_sh.at[slotidx_v.at[0]], add=True)
        return carry

    for seg in range(NSEG):
        base = w * EPW + seg * SEGE
        pltpu.sync_copy(dst_hbm.at[pl.ds(base, SEGE)], dst_v)
        pltpu.sync_copy(src_hbm.at[pl.ds(base, SEGE)], src_v)

        def _scan(i, nvec):
            d = dst_v[pl.ds(i * 16, 16)]
            sv = src_v[pl.ds(i * 16, 16)]
            slot = plsc.load_gather(map_v, [d])
            m = slot >= 0
            plsc.addupdate_scatter(
                deg_v,
                [lax.shift_right_logical(slot, 7),
                 lax.bitwise_and(slot, 127)],
                ones16, mask=m)
            inc = jnp.where(m, 1, 0).astype(jnp.int32)
            pos = nvec + plsc.cumsum(inc) - 1
            r = lax.shift_right_logical(pos, 7)
            cc = lax.bitwise_and(pos, 127)
            plsc.store_scatter(pair_v, [r, cc], sv * PACK + slot, mask=m)
            return nvec + plsc.all_reduce_population_count(m)

        nvec = lax.fori_loop(0, SVECS, _scan, jnp.zeros((16,), jnp.int32))
        n_valid = jnp.max(nvec)

        for k in range(CHUNK // 16):
            p = n_valid + k * 16 + iota16
            r = lax.shift_right_logical(p, 7)
            cc = lax.bitwise_and(p, 127)
            plsc.store_scatter(pair_v, [r, cc], dummy_pair)

        nch = lax.div(n_valid + (CHUNK - 1), jnp.int32(CHUNK))
        lax.fori_loop(0, nch, _chunk, 0)

    pltpu.sync_copy(deg_v, degstage_sh.at[s])

    pltpu.async_copy(
        x_sh.at[nodes_v.at[pl.ds(w * 32, 32)]],
        rows_v.at[pl.ds(0, 32)], dsem).wait()
    pltpu.sync_copy(rows_v.at[pl.ds(0, 32)], xg_hbm.at[pl.ds(w * 32, 32)])

    plsc.subcore_barrier()

    r0 = lax.shift_right_logical(s, 1)
    cb = lax.bitwise_and(s, 1) * 64
    pltpu.sync_copy(degstage_sh.at[:, r0, :], degtmp_v)

    def _red(t, a):
        return (a[0] + degtmp_v[t, pl.ds(cb, 16)],
                a[1] + degtmp_v[t, pl.ds(cb + 16, 16)],
                a[2] + degtmp_v[t, pl.ds(cb + 32, 16)],
                a[3] + degtmp_v[t, pl.ds(cb + 48, 16)])

    z16 = jnp.zeros((16,), jnp.float32)
    a = lax.fori_loop(0, NS, _red, (z16, z16, z16, z16))
    for k in range(4):
        degfin_v[pl.ds(k * 16, 16)] = a[k]
    pltpu.sync_copy(degfin_v, degfinal_sh.at[r0, pl.ds(cb, 64)])

    plsc.subcore_barrier()

    pltpu.sync_copy(degfinal_sh, degall_v)

    def _slots(k, carry):
        nd = nodes_v[pl.ds(s * BPS + k * 16, 16)]
        sl = plsc.load_gather(map_v, [nd])
        slotsel_v[pl.ds(k * 16, 16)] = sl
        degsel_v[pl.ds(k * 16, 16)] = plsc.load_gather(
            degall_v,
            [lax.shift_right_logical(sl, 7), lax.bitwise_and(sl, 127)])
        return carry

    lax.fori_loop(0, BPS // 16, _slots, 0)

    pltpu.async_copy(acc_sh.at[slotsel_v], rows_v.at[pl.ds(0, BPS)],
                     dsem).wait()
    pltpu.sync_copy(rows_v.at[pl.ds(0, BPS)],
                    accpos_hbm.at[c, pl.ds(s * BPS, BPS)])
    pltpu.sync_copy(degsel_v, degpos_hbm.at[c, pl.ds(s * BPS, BPS)])


def _sc_stage(x, src, dst, nodes32):
    mneg = jnp.full((N_NODES,), -1, jnp.int32)
    zacc = jnp.zeros((ACC_ROWS, D_FEAT), jnp.float32)
    zdeg = jnp.zeros((DROWS, DCOLS), jnp.float32)
    mesh = plsc.VectorSubcoreMesh(
        core_axis_name="c", subcore_axis_name="s",
        num_cores=NC, num_subcores=NS)
    return pl.kernel(
        _sc_body,
        out_type=[
            jax.ShapeDtypeStruct((NC, BATCH, D_FEAT), jnp.float32),
            jax.ShapeDtypeStruct((NC, BATCH), jnp.float32),
            jax.ShapeDtypeStruct((BATCH, D_FEAT), jnp.float32),
        ],
        mesh=mesh,
        compiler_params=pltpu.CompilerParams(needs_layout_passes=False),
        scratch_types=[
            pltpu.VMEM((N_NODES,), jnp.int32),
            pltpu.VMEM((BATCH,), jnp.int32),
            pltpu.VMEM((SEGE,), jnp.int32),
            pltpu.VMEM((SEGE,), jnp.int32),
            pltpu.VMEM((PROWS, CHUNK), jnp.int32),
            pltpu.VMEM((CHUNK, D_FEAT), jnp.float32),
            pltpu.VMEM((1, CHUNK), jnp.int32),
            pltpu.VMEM((1, CHUNK), jnp.int32),
            pltpu.VMEM((DROWS, DCOLS), jnp.float32),
            pltpu.VMEM((NS, DCOLS), jnp.float32),
            pltpu.VMEM((64,), jnp.float32),
            pltpu.VMEM((DROWS, DCOLS), jnp.float32),
            pltpu.VMEM((BPS,), jnp.int32),
            pltpu.VMEM((BPS,), jnp.float32),
            pltpu.VMEM_SHARED((N_NODES, D_FEAT), jnp.float32),
            pltpu.VMEM_SHARED((ACC_ROWS, D_FEAT), jnp.float32),
            pltpu.VMEM_SHARED((NS, DROWS, DCOLS), jnp.float32),
            pltpu.VMEM_SHARED((DROWS, DCOLS), jnp.float32),
            pltpu.SemaphoreType.DMA,
        ],
    )(x, src, dst, nodes32, mneg, zacc, zdeg)


def _tc_body(accpos_ref, degpos_ref, xg_ref, wenc_ref, wcls_ref, b_ref,
             out_ref):
    acc = accpos_ref[0] + accpos_ref[1]
    deg = degpos_ref[0] + degpos_ref[1]
    neigh = acc / jnp.maximum(deg, 1.0)[:, None]
    w_self = wenc_ref[:, :D_FEAT]
    w_neigh = wenc_ref[:, D_FEAT:]
    dn = (((1,), (1,)), ((), ()))
    h = lax.dot_general(xg_ref[...], w_self, dn,
                        preferred_element_type=jnp.float32)
    h += lax.dot_general(neigh, w_neigh, dn,
                         preferred_element_type=jnp.float32)
    h = jnp.maximum(h, 0.0)
    out_ref[...] = lax.dot_general(h, wcls_ref[...], dn,
                                   preferred_element_type=jnp.float32) + b_ref[...]


def _tc_stage(accpos, degpos, xg, W_enc, W_cls, b2):
    return pl.pallas_call(
        _tc_body,
        out_shape=jax.ShapeDtypeStruct((BATCH, NUM_CLASSES), jnp.float32),
    )(accpos, degpos, xg, W_enc, W_cls, b2)


def kernel(x, edge_index, nodes, W_enc, W_cls, b_cls):
    src = edge_index[0].astype(jnp.int32)
    dst = edge_index[1].astype(jnp.int32)
    nodes32 = nodes.astype(jnp.int32)
    accpos, degpos, xg = _sc_stage(x, src, dst, nodes32)
    b2 = b_cls.reshape(1, NUM_CLASSES)
    return _tc_stage(accpos, degpos, xg, W_enc, W_cls, b2)


# speedup vs baseline: 1.5469x; 1.0631x over previous
import jax
import jax.numpy as jnp
from jax import lax
from jax.experimental import pallas as pl
from jax.experimental.pallas import tpu as pltpu
from jax.experimental.pallas import tpu_sc as plsc

N_NODES = 10000
N_EDGES = 320000
D_FEAT = 128
EMBED_DIM = 128
NUM_CLASSES = 40
BATCH = 1024

NC = 2
NS = 16
NW = NC * NS
EPW = N_EDGES // NW
NSEG = 5
SEGE = EPW // NSEG
SVECS = SEGE // 16
CHUNK = 128
PROWS = SEGE // CHUNK + 2
ACC_ROWS = BATCH + CHUNK
ZROWS = ACC_ROWS // NS
XROWS = 632
XTAIL = N_NODES - (NS - 1) * XROWS
DROWS = 8
DCOLS = 128
BPS = BATCH // NS
PACK = 2048


def _sc_body(x_hbm, src_hbm, dst_hbm, nodes_hbm, mneg_hbm, zacc_hbm,
             zdeg_hbm, accpos_hbm, degpos_hbm, xg_hbm,
             map_v, nodes_v, dst_v, src_v, pair_v, rows_v, srcidx_v,
             slotidx_v, deg_v, degtmp_v, degfin_v, degall_v, slotsel_v,
             degsel_v, x_sh, acc_sh, degstage_sh, degfinal_sh, dsem):
    c = lax.axis_index("c")
    s = lax.axis_index("s")
    w = s * NC + c
    iota16 = lax.iota(jnp.int32, 16)

    pltpu.sync_copy(nodes_hbm, nodes_v)
    pltpu.sync_copy(mneg_hbm, map_v)
    pltpu.sync_copy(zdeg_hbm, deg_v)

    pltpu.sync_copy(zacc_hbm.at[pl.ds(s * ZROWS, ZROWS)],
                    acc_sh.at[pl.ds(s * ZROWS, ZROWS)])
    @pl.when(s < NS - 1)
    def _xstage():
        pltpu.sync_copy(x_hbm.at[pl.ds(s * XROWS, XROWS)],
                        x_sh.at[pl.ds(s * XROWS, XROWS)])

    @pl.when(s == NS - 1)
    def _xstage_tail():
        pltpu.sync_copy(x_hbm.at[pl.ds((NS - 1) * XROWS, XTAIL)],
                        x_sh.at[pl.ds((NS - 1) * XROWS, XTAIL)])

    def _mapbuild(i, carry):
        nd = nodes_v[pl.ds(i * 16, 16)]
        plsc.store_scatter(map_v, [nd], i * 16 + iota16)
        return carry

    lax.fori_loop(0, BATCH // 16, _mapbuild, 0)

    plsc.subcore_barrier()

    ones16 = jnp.ones((16,), jnp.float32)
    dummy_pair = jnp.full((16,), BATCH, jnp.int32)

    def _chunk(j, carry):
        for k in range(CHUNK // 16):
            pv = pair_v[j, pl.ds(k * 16, 16)]
            srcidx_v[0, pl.ds(k * 16, 16)] = lax.shift_right_logical(pv, 11)
            slotidx_v[0, pl.ds(k * 16, 16)] = lax.bitwise_and(
                pv, PACK - 1)
        pltpu.async_copy(x_sh.at[srcidx_v.at[0]], rows_v, dsem).wait()
        pltpu.sync_copy(rows_v, acc_sh.at[slotidx_v.at[0]], add=True)
        return carry

    tail = jnp.int32(0)
    for seg in range(NSEG):
        base = w * EPW + seg * SEGE
        pltpu.sync_copy(dst_hbm.at[pl.ds(base, SEGE)], dst_v)
        pltpu.sync_copy(src_hbm.at[pl.ds(base, SEGE)], src_v)

        def _scan(i, nvec):
            d = dst_v[pl.ds(i * 16, 16)]
            sv = src_v[pl.ds(i * 16, 16)]
            slot = plsc.load_gather(map_v, [d])
            m = slot >= 0
            plsc.addupdate_scatter(
                deg_v,
                [lax.shift_right_logical(slot, 7),
                 lax.bitwise_and(slot, 127)],
                ones16, mask=m)
            inc = jnp.where(m, 1, 0).astype(jnp.int32)
            pos = nvec + plsc.cumsum(inc) - 1
            r = lax.shift_right_logical(pos, 7)
            cc = lax.bitwise_and(pos, 127)
            plsc.store_scatter(pair_v, [r, cc], sv * PACK + slot, mask=m)
            return nvec + plsc.all_reduce_population_count(m)

        nvec = lax.fori_loop(
            0, SVECS, _scan, jnp.full((16,), tail, jnp.int32))
        n_valid = jnp.max(nvec)

        nfull = lax.shift_right_logical(n_valid, 7)
        lax.fori_loop(0, nfull, _chunk, 0)
        tail = lax.bitwise_and(n_valid, 127)

        @pl.when(nfull > 0)
        def _movetail():
            for k in range(CHUNK // 16):
                pv = pair_v[nfull, pl.ds(k * 16, 16)]
                pair_v[0, pl.ds(k * 16, 16)] = pv

    for k in range(CHUNK // 16):
        p = tail + k * 16 + iota16
        r = lax.shift_right_logical(p, 7)
        cc = lax.bitwise_and(p, 127)
        plsc.store_scatter(pair_v, [r, cc], dummy_pair)

    @pl.when(tail > 0)
    def _lastchunk():
        _chunk(0, 0)

    pltpu.sync_copy(deg_v, degstage_sh.at[s])

    pltpu.async_copy(
        x_sh.at[nodes_v.at[pl.ds(w * 32, 32)]],
        rows_v.at[pl.ds(0, 32)], dsem).wait()
    pltpu.sync_copy(rows_v.at[pl.ds(0, 32)], xg_hbm.at[pl.ds(w * 32, 32)])

    plsc.subcore_barrier()

    r0 = lax.shift_right_logical(s, 1)
    cb = lax.bitwise_and(s, 1) * 64
    pltpu.sync_copy(degstage_sh.at[:, r0, :], degtmp_v)

    def _red(t, a):
        return (a[0] + degtmp_v[t, pl.ds(cb, 16)],
                a[1] + degtmp_v[t, pl.ds(cb + 16, 16)],
                a[2] + degtmp_v[t, pl.ds(cb + 32, 16)],
                a[3] + degtmp_v[t, pl.ds(cb + 48, 16)])

    z16 = jnp.zeros((16,), jnp.float32)
    a = lax.fori_loop(0, NS, _red, (z16, z16, z16, z16))
    for k in range(4):
        degfin_v[pl.ds(k * 16, 16)] = a[k]
    pltpu.sync_copy(degfin_v, degfinal_sh.at[r0, pl.ds(cb, 64)])

    plsc.subcore_barrier()

    pltpu.sync_copy(degfinal_sh, degall_v)

    def _slots(k, carry):
        nd = nodes_v[pl.ds(s * BPS + k * 16, 16)]
        sl = plsc.load_gather(map_v, [nd])
        slotsel_v[pl.ds(k * 16, 16)] = sl
        degsel_v[pl.ds(k * 16, 16)] = plsc.load_gather(
            degall_v,
            [lax.shift_right_logical(sl, 7), lax.bitwise_and(sl, 127)])
        return carry

    lax.fori_loop(0, BPS // 16, _slots, 0)

    pltpu.async_copy(acc_sh.at[slotsel_v], rows_v.at[pl.ds(0, BPS)],
                     dsem).wait()
    pltpu.sync_copy(rows_v.at[pl.ds(0, BPS)],
                    accpos_hbm.at[c, pl.ds(s * BPS, BPS)])
    pltpu.sync_copy(degsel_v, degpos_hbm.at[c, pl.ds(s * BPS, BPS)])


def _sc_stage(x, src, dst, nodes32):
    mneg = jnp.full((N_NODES,), -1, jnp.int32)
    zacc = jnp.zeros((ACC_ROWS, D_FEAT), jnp.float32)
    zdeg = jnp.zeros((DROWS, DCOLS), jnp.float32)
    mesh = plsc.VectorSubcoreMesh(
        core_axis_name="c", subcore_axis_name="s",
        num_cores=NC, num_subcores=NS)
    return pl.kernel(
        _sc_body,
        out_type=[
            jax.ShapeDtypeStruct((NC, BATCH, D_FEAT), jnp.float32),
            jax.ShapeDtypeStruct((NC, BATCH), jnp.float32),
            jax.ShapeDtypeStruct((BATCH, D_FEAT), jnp.float32),
        ],
        mesh=mesh,
        compiler_params=pltpu.CompilerParams(needs_layout_passes=False),
        scratch_types=[
            pltpu.VMEM((N_NODES,), jnp.int32),
            pltpu.VMEM((BATCH,), jnp.int32),
            pltpu.VMEM((SEGE,), jnp.int32),
            pltpu.VMEM((SEGE,), jnp.int32),
            pltpu.VMEM((PROWS, CHUNK), jnp.int32),
            pltpu.VMEM((CHUNK, D_FEAT), jnp.float32),
            pltpu.VMEM((1, CHUNK), jnp.int32),
            pltpu.VMEM((1, CHUNK), jnp.int32),
            pltpu.VMEM((DROWS, DCOLS), jnp.float32),
            pltpu.VMEM((NS, DCOLS), jnp.float32),
            pltpu.VMEM((64,), jnp.float32),
            pltpu.VMEM((DROWS, DCOLS), jnp.float32),
            pltpu.VMEM((BPS,), jnp.int32),
            pltpu.VMEM((BPS,), jnp.float32),
            pltpu.VMEM_SHARED((N_NODES, D_FEAT), jnp.float32),
            pltpu.VMEM_SHARED((ACC_ROWS, D_FEAT), jnp.float32),
            pltpu.VMEM_SHARED((NS, DROWS, DCOLS), jnp.float32),
            pltpu.VMEM_SHARED((DROWS, DCOLS), jnp.float32),
            pltpu.SemaphoreType.DMA,
        ],
    )(x, src, dst, nodes32, mneg, zacc, zdeg)


def _tc_body(accpos_ref, degpos_ref, xg_ref, wenc_ref, wcls_ref, b_ref,
             out_ref):
    acc = accpos_ref[0] + accpos_ref[1]
    deg = degpos_ref[0] + degpos_ref[1]
    neigh = acc / jnp.maximum(deg, 1.0)[:, None]
    w_self = wenc_ref[:, :D_FEAT]
    w_neigh = wenc_ref[:, D_FEAT:]
    dn = (((1,), (1,)), ((), ()))
    h = lax.dot_general(xg_ref[...], w_self, dn,
                        preferred_element_type=jnp.float32)
    h += lax.dot_general(neigh, w_neigh, dn,
                         preferred_element_type=jnp.float32)
    h = jnp.maximum(h, 0.0)
    out_ref[...] = lax.dot_general(h, wcls_ref[...], dn,
                                   preferred_element_type=jnp.float32) + b_ref[...]


def _tc_stage(accpos, degpos, xg, W_enc, W_cls, b2):
    return pl.pallas_call(
        _tc_body,
        out_shape=jax.ShapeDtypeStruct((BATCH, NUM_CLASSES), jnp.float32),
    )(accpos, degpos, xg, W_enc, W_cls, b2)


def kernel(x, edge_index, nodes, W_enc, W_cls, b_cls):
    src = edge_index[0].astype(jnp.int32)
    dst = edge_index[1].astype(jnp.int32)
    nodes32 = nodes.astype(jnp.int32)
    accpos, degpos, xg = _sc_stage(x, src, dst, nodes32)
    b2 = b_cls.reshape(1, NUM_CLASSES)
    return _tc_stage(accpos, degpos, xg, W_enc, W_cls, b2)
